# baseline (speedup 1.0000x reference)
import functools

import jax
import jax.numpy as jnp
from jax import lax
from jax.experimental import pallas as pl
from jax.experimental.pallas import tpu as pltpu
from jax.experimental.pallas import tpu_sc as plsc

NBUF = 3
NOUT = 2
CHUNK = 128


@functools.lru_cache(maxsize=None)
def _build_sc_embed(B, L, D):
    info = plsc.get_sparse_core_info()
    NC, NS = info.num_cores, info.num_subcores
    NW = NC * NS
    assert D == 64 and B == NW * CHUNK
    period = NBUF * NOUT
    assert (L - 2) % period == 0 and L >= period + 2
    n_packs = (L - 2) // period
    mesh = plsc.VectorSubcoreMesh(core_axis_name="c", subcore_axis_name="s")

    @functools.partial(
        pl.kernel,
        mesh=mesh,
        compiler_params=pltpu.CompilerParams(needs_layout_passes=False),
        out_type=jax.ShapeDtypeStruct((L, D, B), jnp.float32),
        scratch_types=(
            [pltpu.VMEM((L, CHUNK), jnp.int32),
             pltpu.VMEM((NBUF, CHUNK), jnp.int32),
             pltpu.VMEM((L, D), jnp.float32)]
            + [pltpu.VMEM((CHUNK, 2 * D), jnp.float32) for _ in range(NBUF)]
            + [pltpu.VMEM((D, CHUNK), jnp.float32) for _ in range(NOUT)]
            + [pltpu.SemaphoreType.DMA for _ in range(NBUF + NOUT)]
        ),
    )
    def embed(x_hbm, tok_hbm, pos_hbm, out_hbm, idx_v, idx2, pos_v, *refs):
        gath = refs[:NBUF]
        outb = refs[NBUF:NBUF + NOUT]
        gsem = refs[NBUF + NOUT:2 * NBUF + NOUT]
        ssem = refs[2 * NBUF + NOUT:]
        wid = lax.axis_index("s") * NC + lax.axis_index("c")
        col = wid * CHUNK

        pltpu.sync_copy(x_hbm.at[:, pl.ds(col, CHUNK)], idx_v)
        pltpu.sync_copy(pos_hbm, pos_v)

        def prep_and_gather(l, b):
            def sh(q, c):
                sl = pl.ds(q * 16, 16)
                idx2[b, sl] = lax.shift_right_logical(idx_v[l, sl], 1)
                return c
            lax.fori_loop(0, CHUNK // 16, sh, 0, unroll=8)
            pltpu.make_async_copy(
                tok_hbm.at[idx2.at[b]], gath[b], gsem[b]).start()

        def wait_gather(b):
            pltpu.make_async_copy(
                tok_hbm.at[idx2.at[b]], gath[b], gsem[b]).wait()

        def start_store(l, o):
            pltpu.make_async_copy(
                outb[o], out_hbm.at[l, :, pl.ds(col, CHUNK)], ssem[o]).start()

        def wait_store(o):
            pltpu.make_async_copy(
                outb[o], out_hbm.at[0, :, pl.ds(col, CHUNK)], ssem[o]).wait()

        def compute(l, b, o):
            lvec = jnp.full((16,), l, jnp.int32)
            rows, pcols = [], []
            for grp in range(CHUNK // 16):
                tvec = idx_v[l, pl.ds(grp * 16, 16)]
                rows.append(lax.iota(jnp.int32, 16) + (grp * 16))
                pcols.append(lax.shift_left(tvec & 1, 6))

            def dim_body(d, c):
                dvec = jnp.full((16,), d, jnp.int32)
                posv = plsc.load_gather(pos_v, [lvec, dvec])
                for grp in range(CHUNK // 16):
                    vals = plsc.load_gather(gath[b], [rows[grp], pcols[grp] + dvec])
                    outb[o][d, pl.ds(grp * 16, 16)] = vals + posv
                return c

            lax.fori_loop(0, D, dim_body, 0, unroll=2)

        def body(l, b, o, prefetch, store_wait):
            if prefetch:
                prep_and_gather(l + 2, (b + 2) % NBUF)
            wait_gather(b)
            if store_wait:
                wait_store(o)
            compute(l, b, o)
            start_store(l, o)

        prep_and_gather(0, 0)
        prep_and_gather(1, 1)
        body(0, 0, 0, True, False)
        body(1, 1, 1, True, False)

        def pack(pk, c):
            l0 = pk * period + 2
            for j in range(period):
                body(l0 + j, (2 + j) % NBUF, j % NOUT, True, True)
            return c

        lax.fori_loop(0, n_packs - 1, pack, 0)

        l0 = (n_packs - 1) * period + 2
        for j in range(period):
            body(l0 + j, (2 + j) % NBUF, j % NOUT, l0 + j + 2 < L, True)
        for o in range(NOUT):
            wait_store(o)

    return embed


def kernel(x, token_table, pos_table):
    B, L = x.shape
    V, D = token_table.shape
    x_t = jnp.transpose(x.astype(jnp.int32))
    tok2 = token_table.reshape(V // 2, 2 * D)
    out = _build_sc_embed(B, L, D)(x_t, tok2, pos_table)
    return jnp.transpose(out, (2, 0, 1))


# speedup vs baseline: 1.2069x; 1.2069x over previous
import functools

import jax
import jax.numpy as jnp
from jax import lax
from jax.experimental import pallas as pl
from jax.experimental.pallas import tpu as pltpu
from jax.experimental.pallas import tpu_sc as plsc

NBUF = 3
NOUT = 2
CHUNK = 128


@functools.lru_cache(maxsize=None)
def _build_sc_embed(B, L, D):
    info = plsc.get_sparse_core_info()
    NC, NS = info.num_cores, info.num_subcores
    NW = NC * NS
    assert D == 64 and B == NW * CHUNK
    period = NBUF * NOUT
    assert (L - 2) % period == 0 and L >= period + 2
    n_packs = (L - 2) // period
    mesh = plsc.VectorSubcoreMesh(core_axis_name="c", subcore_axis_name="s")

    @functools.partial(
        pl.kernel,
        mesh=mesh,
        compiler_params=pltpu.CompilerParams(needs_layout_passes=False),
        out_type=jax.ShapeDtypeStruct((L, D, B), jnp.float32),
        scratch_types=(
            [pltpu.VMEM((L, CHUNK), jnp.int32),
             pltpu.VMEM((NBUF, CHUNK), jnp.int32),
             pltpu.VMEM((L, D), jnp.float32)]
            + [pltpu.VMEM((CHUNK, 2 * D), jnp.float32) for _ in range(NBUF)]
            + [pltpu.VMEM((D, CHUNK), jnp.float32) for _ in range(NOUT)]
            + [pltpu.SemaphoreType.DMA for _ in range(NBUF + NOUT)]
        ),
    )
    def embed(x_hbm, tok_hbm, pos_hbm, out_hbm, idx_v, idx2, pos_v, *refs):
        gath = refs[:NBUF]
        outb = refs[NBUF:NBUF + NOUT]
        gsem = refs[NBUF + NOUT:2 * NBUF + NOUT]
        ssem = refs[2 * NBUF + NOUT:]
        wid = lax.axis_index("s") * NC + lax.axis_index("c")
        col = wid * CHUNK

        pltpu.sync_copy(x_hbm.at[:, pl.ds(col, CHUNK)], idx_v)
        pltpu.sync_copy(pos_hbm, pos_v)

        def prep_and_gather(l, b):
            def sh(q, c):
                sl = pl.ds(q * 16, 16)
                idx2[b, sl] = lax.shift_right_logical(idx_v[l, sl], 1)
                return c
            lax.fori_loop(0, CHUNK // 16, sh, 0, unroll=8)
            pltpu.make_async_copy(
                tok_hbm.at[idx2.at[b]], gath[b], gsem[b]).start()

        def wait_gather(b):
            pltpu.make_async_copy(
                tok_hbm.at[idx2.at[b]], gath[b], gsem[b]).wait()

        def start_store(l, o):
            pltpu.make_async_copy(
                outb[o], out_hbm.at[l, :, pl.ds(col, CHUNK)], ssem[o]).start()

        def wait_store(o):
            pltpu.make_async_copy(
                outb[o], out_hbm.at[0, :, pl.ds(col, CHUNK)], ssem[o]).wait()

        def compute(l, b, o):
            ng = CHUNK // 16
            lvec = jnp.full((16,), l, jnp.int32)
            rows, pcols = [], []
            for grp in range(ng):
                tvec = idx_v[l, pl.ds(grp * 16, 16)]
                rows.append(lax.iota(jnp.int32, 16) + (grp * 16))
                pcols.append(lax.shift_left(tvec & 1, 6))

            def dim_body(d, c):
                dvec = jnp.full((16,), d, jnp.int32)
                posv = plsc.load_gather(pos_v, [lvec, dvec])
                vals = [plsc.load_gather(gath[b], [rows[g], pcols[g] + dvec])
                        for g in range(ng)]
                for g in range(ng):
                    outb[o][d, pl.ds(g * 16, 16)] = vals[g] + posv
                return c

            lax.fori_loop(0, D, dim_body, 0, unroll=4)

        def body(l, b, o, prefetch, store_wait):
            if prefetch:
                prep_and_gather(l + 2, (b + 2) % NBUF)
            wait_gather(b)
            if store_wait:
                wait_store(o)
            compute(l, b, o)
            start_store(l, o)

        prep_and_gather(0, 0)
        prep_and_gather(1, 1)
        body(0, 0, 0, True, False)
        body(1, 1, 1, True, False)

        def pack(pk, c):
            l0 = pk * period + 2
            for j in range(period):
                body(l0 + j, (2 + j) % NBUF, j % NOUT, True, True)
            return c

        lax.fori_loop(0, n_packs - 1, pack, 0)

        l0 = (n_packs - 1) * period + 2
        for j in range(period):
            body(l0 + j, (2 + j) % NBUF, j % NOUT, l0 + j + 2 < L, True)
        for o in range(NOUT):
            wait_store(o)

    return embed


def kernel(x, token_table, pos_table):
    B, L = x.shape
    V, D = token_table.shape
    x_t = jnp.transpose(x.astype(jnp.int32))
    tok2 = token_table.reshape(V // 2, 2 * D)
    out = _build_sc_embed(B, L, D)(x_t, tok2, pos_table)
    return jnp.transpose(out, (2, 0, 1))


# speedup vs baseline: 1.5739x; 1.3040x over previous
import functools

import jax
import jax.numpy as jnp
from jax import lax
from jax.experimental import pallas as pl
from jax.experimental.pallas import tpu as pltpu
from jax.experimental.pallas import tpu_sc as plsc

NBUF = 4


@functools.lru_cache(maxsize=None)
def _build_sc_embed(B, L, D):
    info = plsc.get_sparse_core_info()
    NC, NS = info.num_cores, info.num_subcores
    NW = NC * NS
    BL = B * L
    CHUNK = L // 2
    assert L % 2 == 0 and D % 16 == 0
    assert BL % (NW * L) == 0
    seq_w = B // NW
    per_w = BL // NW
    n_chunks = per_w // CHUNK
    assert n_chunks % NBUF == 0 and n_chunks >= 2 * NBUF
    n_quads = n_chunks // NBUF
    mesh = plsc.VectorSubcoreMesh(core_axis_name="c", subcore_axis_name="s")

    @functools.partial(
        pl.kernel,
        mesh=mesh,
        compiler_params=pltpu.CompilerParams(use_tc_tiling_on_sc=False),
        out_type=jax.ShapeDtypeStruct((B, L, D), jnp.float32),
        scratch_types=(
            [pltpu.VMEM((n_chunks, CHUNK), jnp.int32),
             pltpu.VMEM((L, D), jnp.float32)]
            + [pltpu.VMEM((CHUNK, D), jnp.float32) for _ in range(NBUF)]
            + [pltpu.SemaphoreType.DMA for _ in range(2 * NBUF)]
        ),
    )
    def embed(x_hbm, tok_hbm, pos_hbm, out_hbm, idx_v, pos_v, *bufs_and_sems):
        rows = bufs_and_sems[:NBUF]
        gsem = bufs_and_sems[NBUF:2 * NBUF]
        ssem = bufs_and_sems[2 * NBUF:]
        wid = lax.axis_index("s") * NC + lax.axis_index("c")
        seq0 = wid * seq_w

        pltpu.sync_copy(x_hbm.at[wid], idx_v)
        pltpu.sync_copy(pos_hbm, pos_v)

        def start_gather(g, b):
            pltpu.make_async_copy(
                tok_hbm.at[idx_v.at[g]], rows[b], gsem[b]).start()

        def wait_gather(g, b):
            pltpu.make_async_copy(
                tok_hbm.at[idx_v.at[g]], rows[b], gsem[b]).wait()

        def start_store(g, b, half):
            pltpu.make_async_copy(
                rows[b],
                out_hbm.at[seq0 + lax.div(g, 2), pl.ds(half * CHUNK, CHUNK)],
                ssem[b]).start()

        def wait_store(b):
            pltpu.make_async_copy(
                rows[b], out_hbm.at[seq0, pl.ds(0, CHUNK)], ssem[b]).wait()

        def add_pos(b, half):
            prow = half * CHUNK

            def row_body(i, c):
                for q in range(D // 16):
                    sl = pl.ds(q * 16, 16)
                    plsc.addupdate(rows[b].at[i, sl], pos_v[prow + i, sl])
                return c

            lax.fori_loop(0, CHUNK, row_body, 0, unroll=4)

        def chunk_body(g, b, half, prefetch, prefetch_wait):
            wait_gather(g, b)
            add_pos(b, half)
            start_store(g, b, half)
            if prefetch:
                nb = (b + NBUF - 1) % NBUF
                if prefetch_wait:
                    wait_store(nb)
                start_gather(g + NBUF - 1, nb)

        for b in range(NBUF - 1):
            start_gather(b, b)
        chunk_body(0, 0, 0, True, False)
        for b in range(1, NBUF):
            chunk_body(b, b, b % 2, True, True)

        def quad(p, c):
            g0 = p * NBUF
            for b in range(NBUF):
                chunk_body(g0 + b, b, b % 2, True, True)
            return c

        lax.fori_loop(1, n_quads - 1, quad, 0)

        g0 = (n_quads - 1) * NBUF
        chunk_body(g0, 0, 0, True, True)
        for b in range(1, NBUF):
            chunk_body(g0 + b, b, b % 2, False, False)
        for b in range(NBUF):
            wait_store(b)

    return embed


def kernel(x, token_table, pos_table):
    B, L = x.shape
    D = token_table.shape[1]
    BL = B * L
    info = plsc.get_sparse_core_info()
    NW = info.num_cores * info.num_subcores
    CHUNK = L // 2
    x_r = x.astype(jnp.int32).reshape(NW, BL // (NW * CHUNK), CHUNK)
    return _build_sc_embed(B, L, D)(x_r, token_table, pos_table)
